# baseline (speedup 1.0000x reference)
import jax
import jax.numpy as jnp
from jax import lax
from jax.experimental import pallas as pl
from jax.experimental.pallas import tpu as pltpu
from jax.experimental.pallas import tpu_sc as plsc

N_FIELDS = 26
VOCAB = 100000
D_MODEL = 64
BATCH = 4096
LANES = 16
NC, NS = 2, 16
NW = NC * NS
B_TOTAL = N_FIELDS * BATCH
CH = 128
CPW = B_TOTAL // (NW * CH)


BPW = CPW * CH


def _body(xs_hbm, tab_hbm, out_hbm, idx_v, rows_v, sem):
    wid = lax.axis_index("s") * NC + lax.axis_index("c")
    base = wid * BPW
    pltpu.sync_copy(xs_hbm.at[pl.ds(base, BPW)], idx_v)
    for k in range(BPW // LANES):
        pos = lax.iota(jnp.int32, 16) + (base + k * LANES)
        fld = lax.shift_right_logical(pos, 12)
        sl = pl.ds(k * LANES, LANES)
        idx_v[sl] = idx_v[sl] + fld * VOCAB
    for ch in range(CPW):
        pltpu.async_copy(
            tab_hbm.at[idx_v.at[pl.ds(ch * CH, CH)]], rows_v, sem).wait()
        pltpu.sync_copy(rows_v, out_hbm.at[pl.ds(base + ch * CH, CH)])


def kernel(xs, tables):
    xs_flat = xs.reshape(B_TOTAL)
    tab = tables.reshape(N_FIELDS * VOCAB, D_MODEL)
    fn = pl.kernel(
        _body,
        mesh=plsc.VectorSubcoreMesh(core_axis_name="c", subcore_axis_name="s"),
        compiler_params=pltpu.CompilerParams(use_tc_tiling_on_sc=False),
        out_type=jax.ShapeDtypeStruct((B_TOTAL, D_MODEL), jnp.float32),
        scratch_types=[
            pltpu.VMEM((BPW,), jnp.int32),
            pltpu.VMEM((CH, D_MODEL), jnp.float32),
            pltpu.SemaphoreType.DMA,
        ],
    )
    out = fn(xs_flat, tab)
    return out.reshape(N_FIELDS, BATCH, D_MODEL)
